# Initial kernel scaffold; baseline (speedup 1.0000x reference)
import jax
import jax.numpy as jnp
from jax.experimental import pallas as pl


def kernel(x, edge_index, batch, W1, b1, W2, b2, Wfc, bfc):
    raise NotImplementedError("write your pallas kernel here")


# speedup vs baseline: 10.5395x; 10.5395x over previous
import functools

import jax
import jax.numpy as jnp
from jax import lax
from jax.experimental import pallas as pl
from jax.experimental.pallas import tpu as pltpu
from jax.experimental.pallas import tpu_sc as plsc

F32 = jnp.float32
NCORES = 2
NSUB = 16
CK = 512
CKR = CK // 128
TBL = 16
HW = 64
G_SEG = 64
TRASH = 256


def _round_up(v, m):
    return (v + m - 1) // m * m


def _clamp_to_range(idx_ref, base, bound, trash, ckr=CKR):
    for r in range(ckr):
        for t in range(8):
            v = idx_ref[r, pl.ds(t * 16, 16)]
            rel = v - base
            m = (rel >= 0) & (rel < bound)
            idx_ref[r, pl.ds(t * 16, 16)] = jnp.where(
                m, rel, jnp.broadcast_to(trash, (16,)))


def _deg_kernel(npad, erows, interpret=False):
    nh = npad // 2
    nacc = nh + TRASH
    rz = nacc // NSUB
    rw = nh // NSUB
    chunks = erows // (NSUB * CKR)
    mesh = plsc.VectorSubcoreMesh(core_axis_name="c", subcore_axis_name="s",
                                  num_cores=NCORES, num_subcores=NSUB)

    @functools.partial(
        pl.kernel,
        out_type=jax.ShapeDtypeStruct((npad, TBL), F32),
        mesh=mesh,
        scratch_types=[
            pltpu.VMEM((CKR, 128), jnp.int32),
            pltpu.VMEM((128, TBL), F32),
            pltpu.VMEM_SHARED((nacc, TBL), F32),
        ],
        compiler_params=pltpu.CompilerParams(use_tc_tiling_on_sc=False),
        interpret=interpret,
    )
    def k(dst_hbm, zeros_hbm, out_hbm, didx, ones_v, acc):
        c = lax.axis_index("c")
        s = lax.axis_index("s")

        @pl.loop(0, 128)
        def _(r):
            ones_v[r, :] = jnp.full((TBL,), 1.0, F32)

        pltpu.sync_copy(zeros_hbm.at[pl.ds(s * rz, rz)],
                        acc.at[pl.ds(s * rz, rz)])
        plsc.subcore_barrier()

        @pl.loop(0, chunks)
        def _(kk):
            row0 = (s * chunks + kk) * CKR
            pltpu.sync_copy(dst_hbm.at[pl.ds(row0, CKR)], didx)
            _clamp_to_range(didx, c * nh, nh, nh + s)
            for j in range(CKR):
                pltpu.sync_copy(ones_v.at[pl.ds(0, 128)],
                                acc.at[didx.at[j]], add=True)

        plsc.subcore_barrier()
        pltpu.sync_copy(acc.at[pl.ds(s * rw, rw)],
                        out_hbm.at[pl.ds(c * nh + s * rw, rw)])

    return k


def _agg1_kernel(npad, erows, interpret=False):
    nh = npad // 2
    nacc = nh + TRASH
    rz = nacc // NSUB
    rw = nh // NSUB
    chunks = erows // (NSUB * CKR)
    mesh = plsc.VectorSubcoreMesh(core_axis_name="c", subcore_axis_name="s",
                                  num_cores=NCORES, num_subcores=NSUB)

    @functools.partial(
        pl.kernel,
        out_type=jax.ShapeDtypeStruct((npad, TBL), F32),
        mesh=mesh,
        scratch_types=[
            pltpu.VMEM((CKR, 128), jnp.int32),
            pltpu.VMEM((CKR, 128), jnp.int32),
            pltpu.VMEM((CK, TBL), F32),
            pltpu.VMEM_SHARED((nacc, TBL), F32),
        ],
        compiler_params=pltpu.CompilerParams(use_tc_tiling_on_sc=False),
        interpret=interpret,
    )
    def k(src_hbm, dst_hbm, tbl_hbm, zeros_hbm, out_hbm, sidx, didx, rows,
          acc):
        c = lax.axis_index("c")
        s = lax.axis_index("s")

        pltpu.sync_copy(zeros_hbm.at[pl.ds(s * rz, rz)],
                        acc.at[pl.ds(s * rz, rz)])
        plsc.subcore_barrier()

        @pl.loop(0, chunks)
        def _(kk):
            row0 = (s * chunks + kk) * CKR
            pltpu.sync_copy(src_hbm.at[pl.ds(row0, CKR)], sidx)
            pltpu.sync_copy(dst_hbm.at[pl.ds(row0, CKR)], didx)
            _clamp_to_range(didx, c * nh, nh, nh + s)
            for j in range(CKR):
                pltpu.sync_copy(tbl_hbm.at[sidx.at[j]],
                                rows.at[pl.ds(j * 128, 128)])
                pltpu.sync_copy(rows.at[pl.ds(j * 128, 128)],
                                acc.at[didx.at[j]], add=True)

        plsc.subcore_barrier()
        pltpu.sync_copy(acc.at[pl.ds(s * rw, rw)],
                        out_hbm.at[pl.ds(c * nh + s * rw, rw)])

    return k


def _agg2_kernel(npad, erows, interpret=False):
    nh = npad // 2
    nacc = nh + TRASH
    rz = nacc // NSUB
    rw = nh // NSUB
    cke = 256
    cker = cke // 128
    chunks = erows // (NSUB * cker)
    mesh = plsc.VectorSubcoreMesh(core_axis_name="c", subcore_axis_name="s",
                                  num_cores=NCORES, num_subcores=NSUB)

    @functools.partial(
        pl.kernel,
        out_type=jax.ShapeDtypeStruct((NCORES * npad, HW), F32),
        mesh=mesh,
        scratch_types=[
            pltpu.VMEM((cker, 128), jnp.int32),
            pltpu.VMEM((cker, 128), jnp.int32),
            pltpu.VMEM((cke, HW), F32),
            pltpu.VMEM_SHARED((nacc, HW), F32),
        ],
        compiler_params=pltpu.CompilerParams(use_tc_tiling_on_sc=False),
        interpret=interpret,
    )
    def k(src_hbm, dst_hbm, gh0, gh1, zeros_hbm, out_hbm, sidx, didx, rows,
          acc):
        c = lax.axis_index("c")
        s = lax.axis_index("s")
        gh_refs = [gh0, gh1]

        for cc in range(NCORES):
            gh = gh_refs[cc]

            @pl.when(c == cc)
            def _():
                for h in range(2):
                    pltpu.sync_copy(zeros_hbm.at[pl.ds(s * rz, rz)],
                                    acc.at[pl.ds(s * rz, rz)])
                    plsc.subcore_barrier()

                    @pl.loop(0, chunks)
                    def _(kk):
                        row0 = (s * chunks + kk) * cker
                        pltpu.sync_copy(src_hbm.at[pl.ds(row0, cker)], sidx)
                        pltpu.sync_copy(dst_hbm.at[pl.ds(row0, cker)], didx)
                        _clamp_to_range(didx, h * nh, nh, nh + s, cker)
                        for j in range(cker):
                            pltpu.sync_copy(gh.at[sidx.at[j]],
                                            rows.at[pl.ds(j * 128, 128)])
                            pltpu.sync_copy(rows.at[pl.ds(j * 128, 128)],
                                            acc.at[didx.at[j]], add=True)

                    plsc.subcore_barrier()
                    pltpu.sync_copy(
                        acc.at[pl.ds(s * rw, rw)],
                        out_hbm.at[pl.ds(cc * npad + h * nh + s * rw, rw)])
                    plsc.subcore_barrier()

    return k


def _dinv_call(deg16, xp, npad, nfeat, interpret=False):
    bn = npad // NSUB

    def body(deg_ref, x_ref, dinv_ref, tbl_ref):
        deg = deg_ref[:, 0:1] + 1.0
        dinv = lax.rsqrt(deg)
        dinv_ref[...] = dinv
        gx = dinv * x_ref[...]
        tbl_ref[...] = jnp.concatenate(
            [gx, jnp.zeros((bn, TBL - nfeat), F32)], axis=1)

    return pl.pallas_call(
        body,
        grid=(NSUB,),
        in_specs=[
            pl.BlockSpec((bn, TBL), lambda i: (i, 0)),
            pl.BlockSpec((bn, nfeat), lambda i: (i, 0)),
        ],
        out_specs=[
            pl.BlockSpec((bn, 1), lambda i: (i, 0)),
            pl.BlockSpec((bn, TBL), lambda i: (i, 0)),
        ],
        out_shape=[
            jax.ShapeDtypeStruct((npad, 1), F32),
            jax.ShapeDtypeStruct((npad, TBL), F32),
        ],
        interpret=interpret,
    )(deg16, xp)


def _dense_call(aggx, tbl, dinv, W1, b1, W2, npad, nfeat, interpret=False):
    bn = npad // NSUB

    def body(aggx_ref, tbl_ref, dinv_ref, w1_ref, b1_ref, w2_ref,
             g0_ref, g1_ref):
        dinv = dinv_ref[...]
        ax = dinv * (aggx_ref[:, 0:nfeat] + tbl_ref[:, 0:nfeat])
        h1 = jnp.maximum(
            jnp.dot(ax, w1_ref[...], preferred_element_type=F32)
            + b1_ref[...], 0.0)
        m = jnp.dot(h1, w2_ref[...], preferred_element_type=F32)
        g = dinv * m
        g0_ref[...] = g[:, 0:HW]
        g1_ref[...] = g[:, HW:2 * HW]

    gspec = pl.BlockSpec((bn, HW), lambda i: (i, 0))
    gshape = jax.ShapeDtypeStruct((npad, HW), F32)
    return pl.pallas_call(
        body,
        grid=(NSUB,),
        in_specs=[
            pl.BlockSpec((bn, TBL), lambda i: (i, 0)),
            pl.BlockSpec((bn, TBL), lambda i: (i, 0)),
            pl.BlockSpec((bn, 1), lambda i: (i, 0)),
            pl.BlockSpec((nfeat, 128), lambda i: (0, 0)),
            pl.BlockSpec((1, 128), lambda i: (0, 0)),
            pl.BlockSpec((128, 128), lambda i: (0, 0)),
        ],
        out_specs=[gspec, gspec],
        out_shape=[gshape, gshape],
        interpret=interpret,
    )(aggx, tbl, dinv, W1, b1, W2)


def _final_call(agg, g0, g1, dinv, b2, batch_row, Wfc, bfc, npad,
                interpret=False):
    bn = npad // NSUB
    nb = NSUB

    def body(a0_ref, a1_ref, g0_ref, g1_ref, dinv_ref, b2_ref, batch_ref,
             wfc_ref, bfc_ref, out_ref, pooled_acc, cnt_acc):
        i = pl.program_id(0)

        @pl.when(i == 0)
        def _():
            pooled_acc[...] = jnp.zeros((G_SEG, 128), F32)
            cnt_acc[...] = jnp.zeros((G_SEG, 128), F32)
            out_ref[...] = jnp.zeros((G_SEG, 10), F32)

        dinv = dinv_ref[...]
        h2 = jnp.concatenate(
            [jnp.maximum(dinv * (a0_ref[...] + g0_ref[...])
                         + b2_ref[:, 0:HW], 0.0),
             jnp.maximum(dinv * (a1_ref[...] + g1_ref[...])
                         + b2_ref[:, HW:2 * HW], 0.0)], axis=1)

        seg = batch_ref[...]
        onehot_t = (lax.broadcasted_iota(jnp.int32, (G_SEG, bn), 0)
                    == seg).astype(F32)
        pooled_acc[...] += lax.dot_general(
            onehot_t, h2, (((1,), (0,)), ((), ())),
            preferred_element_type=F32)
        cnt_acc[...] += lax.dot_general(
            onehot_t, jnp.ones((bn, 128), F32), (((1,), (0,)), ((), ())),
            preferred_element_type=F32)

        @pl.when(i == nb - 1)
        def _():
            mean = pooled_acc[...] / jnp.maximum(cnt_acc[...], 1.0)
            out_ref[...] = jnp.dot(mean, wfc_ref[...],
                                   preferred_element_type=F32) + bfc_ref[...]

    hspec = lambda cc: pl.BlockSpec((bn, HW),
                                    lambda i, cc=cc: (cc * NSUB + i, 0))
    gspec = pl.BlockSpec((bn, HW), lambda i: (i, 0))
    return pl.pallas_call(
        body,
        grid=(nb,),
        in_specs=[
            hspec(0), hspec(1),
            gspec, gspec,
            pl.BlockSpec((bn, 1), lambda i: (i, 0)),
            pl.BlockSpec((1, 128), lambda i: (0, 0)),
            pl.BlockSpec((1, bn), lambda i: (0, i)),
            pl.BlockSpec((128, 10), lambda i: (0, 0)),
            pl.BlockSpec((1, 10), lambda i: (0, 0)),
        ],
        out_specs=pl.BlockSpec((G_SEG, 10), lambda i: (0, 0)),
        out_shape=jax.ShapeDtypeStruct((G_SEG, 10), F32),
        scratch_shapes=[
            pltpu.VMEM((G_SEG, 128), F32),
            pltpu.VMEM((G_SEG, 128), F32),
        ],
        interpret=interpret,
    )(agg, agg, g0, g1, dinv, b2, batch_row, Wfc, bfc)


def _run(x, edge_index, batch, W1, b1, W2, b2, Wfc, bfc,
         interpret_sc=False, interpret_tc=False):
    n, nfeat = x.shape
    e = edge_index.shape[1]
    npad = _round_up(n + 1, 2 * NSUB * 128)
    epad = _round_up(e, NSUB * CK)
    erows = epad // 128
    nacc = npad // 2 + TRASH

    src = edge_index[0]
    dst = edge_index[1]
    padlen = epad - e
    fill = jnp.full((padlen,), n, jnp.int32)
    srcr = jnp.concatenate([src, fill]).reshape(erows, 128)
    dstr = jnp.concatenate([dst, fill]).reshape(erows, 128)
    xp = jnp.pad(x, ((0, npad - n), (0, 0)))
    batch_row = jnp.pad(batch, (0, npad - n),
                        constant_values=G_SEG).reshape(1, npad)
    zeros16 = jnp.zeros((nacc, TBL), F32)
    zeros64 = jnp.zeros((nacc, HW), F32)

    deg16 = _deg_kernel(npad, erows, interpret_sc)(dstr, zeros16)
    dinv, tbl = _dinv_call(deg16, xp, npad, nfeat, interpret_tc)
    aggx = _agg1_kernel(npad, erows, interpret_sc)(srcr, dstr, tbl, zeros16)
    g0, g1 = _dense_call(aggx, tbl, dinv, W1, b1.reshape(1, 128), W2,
                         npad, nfeat, interpret_tc)
    agg = _agg2_kernel(npad, erows, interpret_sc)(srcr, dstr, g0, g1, zeros64)
    out = _final_call(agg, g0, g1, dinv, b2.reshape(1, 128), batch_row,
                      Wfc, bfc.reshape(1, 10), npad, interpret_tc)
    return out


def kernel(x, edge_index, batch, W1, b1, W2, b2, Wfc, bfc):
    return _run(x, edge_index, batch, W1, b1, W2, b2, Wfc, bfc)


# speedup vs baseline: 16.3688x; 1.5531x over previous
import functools

import jax
import jax.numpy as jnp
from jax import lax
from jax.experimental import pallas as pl
from jax.experimental.pallas import tpu as pltpu
from jax.experimental.pallas import tpu_sc as plsc

F32 = jnp.float32
NCORES = 2
NSUB = 16
CK = 512
CKR = CK // 128
TBL = 16
HW = 64
G_SEG = 64
TRASH = 16


def _round_up(v, m):
    return (v + m - 1) // m * m


def _clamp_to_range(idx_ref, base, bound, trash, ckr=CKR):
    for r in range(ckr):
        for t in range(8):
            v = idx_ref[r, pl.ds(t * 16, 16)]
            rel = v - base
            m = (rel >= 0) & (rel < bound)
            idx_ref[r, pl.ds(t * 16, 16)] = jnp.where(
                m, rel, jnp.broadcast_to(trash, (16,)))


def _edge_sweep(src_hbm, dst_hbm, tbl, acc, sidx, didx, r0, r1, sem0, sem1,
                s, groups, base, nh, trash):
    def fire(j, rbuf, sem):
        pltpu.async_copy(tbl.at[sidx.at[j]], rbuf, sem)

    def wait(j, rbuf, sem):
        pltpu.make_async_copy(tbl.at[sidx.at[j]], rbuf, sem).wait()

    def scat(j, rbuf):
        pltpu.sync_copy(rbuf, acc.at[didx.at[j]], add=True)

    @pl.loop(0, groups)
    def _(g):
        row0 = s * (groups * 8) + g * 8
        pltpu.sync_copy(src_hbm.at[pl.ds(row0, 8)], sidx)
        pltpu.sync_copy(dst_hbm.at[pl.ds(row0, 8)], didx)
        _clamp_to_range(didx, base, nh, trash, 8)
        fire(0, r0, sem0)

        @pl.loop(0, 4)
        def _(p):
            j0 = 2 * p
            fire(j0 + 1, r1, sem1)
            wait(j0, r0, sem0)
            scat(j0, r0)

            @pl.when(p < 3)
            def _():
                fire(j0 + 2, r0, sem0)

            wait(j0 + 1, r1, sem1)
            scat(j0 + 1, r1)


def _deg_kernel(npad, erows, interpret=False):
    nh = npad // 2
    nacc = nh + TRASH
    rz = nacc // NSUB
    rw = nh // NSUB
    chunks = erows // (NSUB * CKR)
    mesh = plsc.VectorSubcoreMesh(core_axis_name="c", subcore_axis_name="s",
                                  num_cores=NCORES, num_subcores=NSUB)

    @functools.partial(
        pl.kernel,
        out_type=jax.ShapeDtypeStruct((npad, TBL), F32),
        mesh=mesh,
        scratch_types=[
            pltpu.VMEM((CKR, 128), jnp.int32),
            pltpu.VMEM((128, TBL), F32),
            pltpu.VMEM_SHARED((nacc, TBL), F32),
        ],
        compiler_params=pltpu.CompilerParams(use_tc_tiling_on_sc=False),
        interpret=interpret,
    )
    def k(dst_hbm, zeros_hbm, out_hbm, didx, ones_v, acc):
        c = lax.axis_index("c")
        s = lax.axis_index("s")

        @pl.loop(0, 128)
        def _(r):
            ones_v[r, :] = jnp.full((TBL,), 1.0, F32)

        pltpu.sync_copy(zeros_hbm.at[pl.ds(s * rz, rz)],
                        acc.at[pl.ds(s * rz, rz)])
        plsc.subcore_barrier()

        @pl.loop(0, chunks)
        def _(kk):
            row0 = (s * chunks + kk) * CKR
            pltpu.sync_copy(dst_hbm.at[pl.ds(row0, CKR)], didx)
            _clamp_to_range(didx, c * nh, nh, nh + s)
            for j in range(CKR):
                pltpu.sync_copy(ones_v.at[pl.ds(0, 128)],
                                acc.at[didx.at[j]], add=True)

        plsc.subcore_barrier()
        pltpu.sync_copy(acc.at[pl.ds(s * rw, rw)],
                        out_hbm.at[pl.ds(c * nh + s * rw, rw)])

    return k


def _agg1_kernel(npad, erows, interpret=False):
    nh = npad // 2
    nacc = nh + TRASH
    rz = nacc // NSUB
    rw = nh // NSUB
    groups = erows // (NSUB * 8)
    mesh = plsc.VectorSubcoreMesh(core_axis_name="c", subcore_axis_name="s",
                                  num_cores=NCORES, num_subcores=NSUB)

    @functools.partial(
        pl.kernel,
        out_type=jax.ShapeDtypeStruct((npad, TBL), F32),
        mesh=mesh,
        scratch_types=[
            pltpu.VMEM((8, 128), jnp.int32),
            pltpu.VMEM((8, 128), jnp.int32),
            pltpu.VMEM((128, TBL), F32),
            pltpu.VMEM((128, TBL), F32),
            pltpu.VMEM_SHARED((nacc, TBL), F32),
            pltpu.SemaphoreType.DMA,
            pltpu.SemaphoreType.DMA,
        ],
        compiler_params=pltpu.CompilerParams(use_tc_tiling_on_sc=False),
        interpret=interpret,
    )
    def k(src_hbm, dst_hbm, tbl_hbm, zeros_hbm, out_hbm, sidx, didx, r0, r1,
          acc, sem0, sem1):
        c = lax.axis_index("c")
        s = lax.axis_index("s")

        pltpu.sync_copy(zeros_hbm.at[pl.ds(s * rz, rz)],
                        acc.at[pl.ds(s * rz, rz)])
        plsc.subcore_barrier()
        _edge_sweep(src_hbm, dst_hbm, tbl_hbm, acc, sidx, didx, r0, r1,
                    sem0, sem1, s, groups, c * nh, nh, nh + s)
        plsc.subcore_barrier()
        pltpu.sync_copy(acc.at[pl.ds(s * rw, rw)],
                        out_hbm.at[pl.ds(c * nh + s * rw, rw)])

    return k


def _agg2_kernel(npad, erows, interpret=False):
    nh = npad // 2
    nacc = nh + TRASH
    rz = nacc // NSUB
    rw = nh // NSUB
    groups = erows // (NSUB * 8)
    mesh = plsc.VectorSubcoreMesh(core_axis_name="c", subcore_axis_name="s",
                                  num_cores=NCORES, num_subcores=NSUB)

    @functools.partial(
        pl.kernel,
        out_type=jax.ShapeDtypeStruct((NCORES * npad, HW), F32),
        mesh=mesh,
        scratch_types=[
            pltpu.VMEM((8, 128), jnp.int32),
            pltpu.VMEM((8, 128), jnp.int32),
            pltpu.VMEM((128, HW), F32),
            pltpu.VMEM((128, HW), F32),
            pltpu.VMEM_SHARED((nacc, HW), F32),
            pltpu.SemaphoreType.DMA,
            pltpu.SemaphoreType.DMA,
        ],
        compiler_params=pltpu.CompilerParams(use_tc_tiling_on_sc=False),
        interpret=interpret,
    )
    def k(src_hbm, dst_hbm, gh0, gh1, zeros_hbm, out_hbm, sidx, didx, r0, r1,
          acc, sem0, sem1):
        c = lax.axis_index("c")
        s = lax.axis_index("s")
        gh_refs = [gh0, gh1]

        for cc in range(NCORES):
            gh = gh_refs[cc]

            @pl.when(c == cc)
            def _():
                for h in range(2):
                    pltpu.sync_copy(zeros_hbm.at[pl.ds(s * rz, rz)],
                                    acc.at[pl.ds(s * rz, rz)])
                    plsc.subcore_barrier()
                    _edge_sweep(src_hbm, dst_hbm, gh, acc, sidx, didx, r0,
                                r1, sem0, sem1, s, groups, h * nh, nh, nh + s)
                    plsc.subcore_barrier()
                    pltpu.sync_copy(
                        acc.at[pl.ds(s * rw, rw)],
                        out_hbm.at[pl.ds(cc * npad + h * nh + s * rw, rw)])
                    plsc.subcore_barrier()

    return k


def _dinv_call(deg16, xp, npad, nfeat, interpret=False):
    bn = npad // NSUB

    def body(deg_ref, x_ref, dinv_ref, tbl_ref):
        deg = deg_ref[:, 0:1] + 1.0
        dinv = lax.rsqrt(deg)
        dinv_ref[...] = dinv
        gx = dinv * x_ref[...]
        tbl_ref[...] = jnp.concatenate(
            [gx, jnp.zeros((bn, TBL - nfeat), F32)], axis=1)

    return pl.pallas_call(
        body,
        grid=(NSUB,),
        in_specs=[
            pl.BlockSpec((bn, TBL), lambda i: (i, 0)),
            pl.BlockSpec((bn, nfeat), lambda i: (i, 0)),
        ],
        out_specs=[
            pl.BlockSpec((bn, 1), lambda i: (i, 0)),
            pl.BlockSpec((bn, TBL), lambda i: (i, 0)),
        ],
        out_shape=[
            jax.ShapeDtypeStruct((npad, 1), F32),
            jax.ShapeDtypeStruct((npad, TBL), F32),
        ],
        interpret=interpret,
    )(deg16, xp)


def _dense_call(aggx, tbl, dinv, W1, b1, W2, npad, nfeat, interpret=False):
    bn = npad // NSUB

    def body(aggx_ref, tbl_ref, dinv_ref, w1_ref, b1_ref, w2_ref,
             g0_ref, g1_ref):
        dinv = dinv_ref[...]
        ax = dinv * (aggx_ref[:, 0:nfeat] + tbl_ref[:, 0:nfeat])
        h1 = jnp.maximum(
            jnp.dot(ax, w1_ref[...], preferred_element_type=F32)
            + b1_ref[...], 0.0)
        m = jnp.dot(h1, w2_ref[...], preferred_element_type=F32)
        g = dinv * m
        g0_ref[...] = g[:, 0:HW]
        g1_ref[...] = g[:, HW:2 * HW]

    gspec = pl.BlockSpec((bn, HW), lambda i: (i, 0))
    gshape = jax.ShapeDtypeStruct((npad, HW), F32)
    return pl.pallas_call(
        body,
        grid=(NSUB,),
        in_specs=[
            pl.BlockSpec((bn, TBL), lambda i: (i, 0)),
            pl.BlockSpec((bn, TBL), lambda i: (i, 0)),
            pl.BlockSpec((bn, 1), lambda i: (i, 0)),
            pl.BlockSpec((nfeat, 128), lambda i: (0, 0)),
            pl.BlockSpec((1, 128), lambda i: (0, 0)),
            pl.BlockSpec((128, 128), lambda i: (0, 0)),
        ],
        out_specs=[gspec, gspec],
        out_shape=[gshape, gshape],
        interpret=interpret,
    )(aggx, tbl, dinv, W1, b1, W2)


def _final_call(agg, g0, g1, dinv, b2, batch_row, Wfc, bfc, npad,
                interpret=False):
    bn = npad // NSUB
    nb = NSUB

    def body(a0_ref, a1_ref, g0_ref, g1_ref, dinv_ref, b2_ref, batch_ref,
             wfc_ref, bfc_ref, out_ref, pooled_acc, cnt_acc):
        i = pl.program_id(0)

        @pl.when(i == 0)
        def _():
            pooled_acc[...] = jnp.zeros((G_SEG, 128), F32)
            cnt_acc[...] = jnp.zeros((G_SEG, 128), F32)
            out_ref[...] = jnp.zeros((G_SEG, 10), F32)

        dinv = dinv_ref[...]
        h2 = jnp.concatenate(
            [jnp.maximum(dinv * (a0_ref[...] + g0_ref[...])
                         + b2_ref[:, 0:HW], 0.0),
             jnp.maximum(dinv * (a1_ref[...] + g1_ref[...])
                         + b2_ref[:, HW:2 * HW], 0.0)], axis=1)

        seg = batch_ref[...]
        onehot_t = (lax.broadcasted_iota(jnp.int32, (G_SEG, bn), 0)
                    == seg).astype(F32)
        pooled_acc[...] += lax.dot_general(
            onehot_t, h2, (((1,), (0,)), ((), ())),
            preferred_element_type=F32)
        cnt_acc[...] += lax.dot_general(
            onehot_t, jnp.ones((bn, 128), F32), (((1,), (0,)), ((), ())),
            preferred_element_type=F32)

        @pl.when(i == nb - 1)
        def _():
            mean = pooled_acc[...] / jnp.maximum(cnt_acc[...], 1.0)
            out_ref[...] = jnp.dot(mean, wfc_ref[...],
                                   preferred_element_type=F32) + bfc_ref[...]

    hspec = lambda cc: pl.BlockSpec((bn, HW),
                                    lambda i, cc=cc: (cc * NSUB + i, 0))
    gspec = pl.BlockSpec((bn, HW), lambda i: (i, 0))
    return pl.pallas_call(
        body,
        grid=(nb,),
        in_specs=[
            hspec(0), hspec(1),
            gspec, gspec,
            pl.BlockSpec((bn, 1), lambda i: (i, 0)),
            pl.BlockSpec((1, 128), lambda i: (0, 0)),
            pl.BlockSpec((1, bn), lambda i: (0, i)),
            pl.BlockSpec((128, 10), lambda i: (0, 0)),
            pl.BlockSpec((1, 10), lambda i: (0, 0)),
        ],
        out_specs=pl.BlockSpec((G_SEG, 10), lambda i: (0, 0)),
        out_shape=jax.ShapeDtypeStruct((G_SEG, 10), F32),
        scratch_shapes=[
            pltpu.VMEM((G_SEG, 128), F32),
            pltpu.VMEM((G_SEG, 128), F32),
        ],
        interpret=interpret,
    )(agg, agg, g0, g1, dinv, b2, batch_row, Wfc, bfc)


def _run(x, edge_index, batch, W1, b1, W2, b2, Wfc, bfc,
         interpret_sc=False, interpret_tc=False):
    n, nfeat = x.shape
    e = edge_index.shape[1]
    npad = _round_up(n + 1, 2 * NSUB * 128)
    epad = _round_up(e, NSUB * 8 * 128)
    erows = epad // 128
    nacc = npad // 2 + TRASH

    src = edge_index[0]
    dst = edge_index[1]
    padlen = epad - e
    fill = jnp.full((padlen,), n, jnp.int32)
    srcr = jnp.concatenate([src, fill]).reshape(erows, 128)
    dstr = jnp.concatenate([dst, fill]).reshape(erows, 128)
    xp = jnp.pad(x, ((0, npad - n), (0, 0)))
    batch_row = jnp.pad(batch, (0, npad - n),
                        constant_values=G_SEG).reshape(1, npad)
    zeros16 = jnp.zeros((nacc, TBL), F32)
    zeros64 = jnp.zeros((nacc, HW), F32)

    deg16 = _deg_kernel(npad, erows, interpret_sc)(dstr, zeros16)
    dinv, tbl = _dinv_call(deg16, xp, npad, nfeat, interpret_tc)
    aggx = _agg1_kernel(npad, erows, interpret_sc)(srcr, dstr, tbl, zeros16)
    g0, g1 = _dense_call(aggx, tbl, dinv, W1, b1.reshape(1, 128), W2,
                         npad, nfeat, interpret_tc)
    agg = _agg2_kernel(npad, erows, interpret_sc)(srcr, dstr, g0, g1, zeros64)
    out = _final_call(agg, g0, g1, dinv, b2.reshape(1, 128), batch_row,
                      Wfc, bfc.reshape(1, 10), npad, interpret_tc)
    return out


def kernel(x, edge_index, batch, W1, b1, W2, b2, Wfc, bfc):
    return _run(x, edge_index, batch, W1, b1, W2, b2, Wfc, bfc)
